# baseline (speedup 1.0000x reference)
import functools

import jax
import jax.numpy as jnp
from jax import lax
from jax.experimental import pallas as pl
from jax.experimental.pallas import tpu as pltpu
from jax.experimental.pallas import tpu_sc as plsc

VOCAB = 256
HIDDEN = 5
CTX = 8
HPAD = 16
FPAD = CTX * HPAD
NW = 32
NCH = 2
LANES = 16


@functools.lru_cache(maxsize=None)
def _make_sc_gather(batch: int, ctx: int, half: int, nhalves: int):
    rows_w = batch // nhalves // NW
    rows_ch = rows_w // NCH
    mesh = plsc.VectorSubcoreMesh(core_axis_name="c", subcore_axis_name="s")

    @functools.partial(
        pl.kernel,
        out_type=jax.ShapeDtypeStruct((batch // nhalves, FPAD), jnp.float32),
        mesh=mesh,
        scratch_types=[
            pltpu.VMEM((rows_w, CTX), jnp.int32),
            pltpu.VMEM((VOCAB, HPAD), jnp.float32),
            pltpu.VMEM((rows_w, FPAD), jnp.float32),
            pltpu.SemaphoreType.DMA,
        ],
        compiler_params=pltpu.CompilerParams(
            use_tc_tiling_on_sc=False, needs_layout_passes=False),
    )
    def sc_gather(idx_hbm, table_hbm, out_hbm, idx_v, table_v, out_v, wsem):
        wid = lax.axis_index("s") * 2 + lax.axis_index("c")
        base = wid * rows_w
        src = half * (batch // nhalves) + base
        pltpu.sync_copy(table_hbm, table_v)
        pltpu.sync_copy(idx_hbm.at[pl.ds(src, rows_w), pl.ds(0, CTX)], idx_v)
        col = lax.iota(jnp.int32, LANES)
        writes = []
        for cb in range(NCH):
            @functools.partial(
                plsc.parallel_loop,
                cb * rows_ch * ctx, (cb + 1) * rows_ch * ctx, unroll=8)
            def body(o):
                r = lax.shift_right_logical(o, 3)
                c = lax.bitwise_and(o, 7)
                rows = plsc.load_gather(
                    idx_v, [lax.broadcast(r, (LANES,)),
                            lax.broadcast(c, (LANES,))])
                vals = plsc.load_gather(table_v, [rows, col])
                out_v[r, pl.ds(c * HPAD, HPAD)] = vals
            writes.append(pltpu.async_copy(
                out_v.at[pl.ds(cb * rows_ch, rows_ch)],
                out_hbm.at[pl.ds(base + cb * rows_ch, rows_ch)],
                wsem))
        for w in writes:
            w.wait()

    return sc_gather


def _dense_body(nsteps_half, emb1_ref, emb2_ref, w_ref, b_ref, out_ref):
    i = pl.program_id(0)

    @pl.when(i < nsteps_half)
    def _():
        out_ref[...] = lax.dot_general(
            emb1_ref[...], w_ref[...], (((1,), (1,)), ((), ())),
            preferred_element_type=jnp.float32) + b_ref[...]

    @pl.when(i >= nsteps_half)
    def _():
        out_ref[...] = lax.dot_general(
            emb2_ref[...], w_ref[...], (((1,), (1,)), ((), ())),
            preferred_element_type=jnp.float32) + b_ref[...]


def _dense(emb1, emb2, w_pad, b2d, batch: int, tile: int):
    nh = batch // 2 // tile
    return pl.pallas_call(
        functools.partial(_dense_body, nh),
        grid=(2 * nh,),
        in_specs=[
            pl.BlockSpec((tile, FPAD), lambda i: (jnp.minimum(i, nh - 1), 0)),
            pl.BlockSpec((tile, FPAD),
                         lambda i: (jnp.maximum(i - nh, 0), 0)),
            pl.BlockSpec((VOCAB, FPAD), lambda i: (0, 0)),
            pl.BlockSpec((1, VOCAB), lambda i: (0, 0)),
        ],
        out_specs=pl.BlockSpec((tile, VOCAB), lambda i: (i, 0)),
        out_shape=jax.ShapeDtypeStruct((batch, VOCAB), jnp.float32),
    )(emb1, emb2, w_pad, b2d)


def kernel(x, embed_table, fc_w, fc_b):
    batch, ctx = x.shape
    vocab, hidden = embed_table.shape

    table_pad = jnp.pad(embed_table, ((0, 0), (0, HPAD - hidden)))
    w_pad = jnp.pad(
        fc_w.reshape(vocab, ctx, hidden), ((0, 0), (0, 0), (0, HPAD - hidden))
    ).reshape(vocab, ctx * HPAD)

    idx2d = jnp.pad(x, ((0, 0), (0, 128 - ctx)))
    emb1 = _make_sc_gather(batch, ctx, 0, 2)(idx2d, table_pad)
    emb2 = _make_sc_gather(batch, ctx, 1, 2)(idx2d, table_pad)
    return _dense(emb1, emb2, w_pad, fc_b.reshape(1, vocab), batch, tile=4096)
